# Initial kernel scaffold; baseline (speedup 1.0000x reference)
import jax
import jax.numpy as jnp
from jax.experimental import pallas as pl


def kernel(x, edge_index, weights, W0, b0, W1, b1):
    raise NotImplementedError("write your pallas kernel here")


# speedup vs baseline: 9.0050x; 9.0050x over previous
import dataclasses
import functools

import jax
import jax.numpy as jnp
from jax import lax
from jax.experimental import pallas as pl
from jax.experimental.pallas import tpu as pltpu
from jax.experimental.pallas import tpu_sc as plsc

NC = 2
NS = 16
NW = NC * NS
LANES = 16
K = 80


def _mesh():
    return plsc.VectorSubcoreMesh(core_axis_name="c", subcore_axis_name="s")


def _sc_compiler_params():
    cp = pltpu.CompilerParams()
    if "needs_layout_passes" in pltpu.CompilerParams.__dataclass_fields__:
        cp = dataclasses.replace(cp, needs_layout_passes=False)
    return cp


def _sc_deg(col, ew, *, n):
    e = col.shape[0]
    epw = e // NW
    ch = epw // K

    @functools.partial(
        pl.kernel,
        mesh=_mesh(),
        out_type=jax.ShapeDtypeStruct((NW * n,), jnp.float32),
        scratch_types=[
            pltpu.VMEM((K,), jnp.int32),
            pltpu.VMEM((K,), jnp.float32),
            pltpu.VMEM((n,), jnp.float32),
        ],
        compiler_params=_sc_compiler_params(),
    )
    def k(col_hbm, ew_hbm, out_hbm, colv, eww, dacc):
        cid = lax.axis_index("c")
        sid = lax.axis_index("s")
        wid = cid * NS + sid
        zero16 = jnp.zeros((LANES,), jnp.float32)

        @pl.loop(0, n // LANES)
        def _(i):
            dacc[pl.ds(i * LANES, LANES)] = zero16

        @pl.loop(0, ch)
        def _(j):
            base = wid * epw + j * K
            pltpu.sync_copy(col_hbm.at[pl.ds(base, K)], colv)
            pltpu.sync_copy(ew_hbm.at[pl.ds(base, K)], eww)

            @pl.loop(0, K // LANES)
            def _(t):
                idx = colv[pl.ds(t * LANES, LANES)]
                v = eww[pl.ds(t * LANES, LANES)]
                plsc.addupdate_scatter(dacc, [idx], v)

        pltpu.sync_copy(dacc, out_hbm.at[pl.ds(wid * n, n)])

    return k(col, ew)


def _sc_msg(row, col, ew, g):
    e = row.shape[0]
    n, d = g.shape
    epw = e // NW
    ch = epw // K
    rpt = n // NS
    zr = 160
    assert n % NS == 0 and rpt % zr == 0 and rpt % 8 == 0

    @functools.partial(
        pl.kernel,
        mesh=_mesh(),
        out_type=jax.ShapeDtypeStruct((NC, n, d), jnp.float32),
        scratch_types=[
            pltpu.VMEM((K,), jnp.int32),
            pltpu.VMEM((K,), jnp.int32),
            pltpu.VMEM((K,), jnp.float32),
            pltpu.VMEM((K, d), jnp.float32),
            pltpu.VMEM((zr, d), jnp.float32),
            pltpu.VMEM_SHARED((n, d), jnp.float32),
            pltpu.SemaphoreType.DMA,
        ],
        compiler_params=_sc_compiler_params(),
    )
    def k(row_hbm, col_hbm, ew_hbm, g_hbm, out_hbm,
          rowv, colv, eww, rows, zbuf, acc, sem):
        cid = lax.axis_index("c")
        sid = lax.axis_index("s")
        zero16 = jnp.zeros((LANES,), jnp.float32)

        @pl.loop(0, zr)
        def _(i):
            for q in range(d // LANES):
                zbuf[i, pl.ds(q * LANES, LANES)] = zero16

        @pl.loop(0, rpt // zr)
        def _(i):
            pltpu.sync_copy(zbuf, acc.at[pl.ds(sid * rpt + i * zr, zr)])

        plsc.subcore_barrier()

        @pl.loop(0, ch)
        def _(j):
            base = (cid * NS + sid) * epw + j * K
            pltpu.sync_copy(row_hbm.at[pl.ds(base, K)], rowv)
            pltpu.sync_copy(col_hbm.at[pl.ds(base, K)], colv)
            pltpu.sync_copy(ew_hbm.at[pl.ds(base, K)], eww)
            pltpu.async_copy(g_hbm.at[rowv], rows, sem).wait()

            @pl.loop(0, K)
            def _(r):
                s = plsc.load_gather(eww, [jnp.full((LANES,), r, jnp.int32)])
                for q in range(d // LANES):
                    rows[r, pl.ds(q * LANES, LANES)] = (
                        rows[r, pl.ds(q * LANES, LANES)] * s
                    )

            pltpu.sync_copy(rows, acc.at[colv], add=True)

        plsc.subcore_barrier()
        pltpu.sync_copy(
            acc.at[pl.ds(sid * rpt, rpt)],
            out_hbm.at[cid, pl.ds(sid * rpt, rpt)],
        )

    return k(row, col, ew, g)


_BR = 1024


def _tc0(x, w0, degp):
    n, d = x.shape

    def body(x_ref, w_ref, degp_ref, g_ref, dinv_ref):
        h = jnp.dot(x_ref[...], w_ref[...], preferred_element_type=jnp.float32)
        ones = jnp.ones((NW, 1), jnp.float32)
        dsum = 1.0 + lax.dot_general(
            degp_ref[...], ones, (((0,), (0,)), ((), ())),
            preferred_element_type=jnp.float32)
        dinv = jnp.where(dsum > 0, lax.rsqrt(dsum), 0.0)
        db = jnp.broadcast_to(dinv, (_BR, d))
        g_ref[...] = h * db
        dinv_ref[...] = db

    return pl.pallas_call(
        body,
        grid=(n // _BR,),
        in_specs=[
            pl.BlockSpec((_BR, d), lambda i: (i, 0)),
            pl.BlockSpec((d, d), lambda i: (0, 0)),
            pl.BlockSpec((NW, _BR), lambda i: (0, i)),
        ],
        out_specs=[
            pl.BlockSpec((_BR, d), lambda i: (i, 0)),
            pl.BlockSpec((_BR, d), lambda i: (i, 0)),
        ],
        out_shape=[
            jax.ShapeDtypeStruct((n, d), jnp.float32),
            jax.ShapeDtypeStruct((n, d), jnp.float32),
        ],
    )(x, w0, degp)


def _tc1(sp, g, dinvb, b, w):
    n, d = g.shape

    def body(sp_ref, g_ref, dinv_ref, b_ref, w_ref, out_ref):
        s = sp_ref[0] + sp_ref[1] + g_ref[...]
        x1 = jnp.maximum(dinv_ref[...] * s + b_ref[...], 0.0)
        h1 = jnp.dot(x1, w_ref[...], preferred_element_type=jnp.float32)
        out_ref[...] = dinv_ref[...] * h1

    return pl.pallas_call(
        body,
        grid=(n // _BR,),
        in_specs=[
            pl.BlockSpec((NC, _BR, d), lambda i: (0, i, 0)),
            pl.BlockSpec((_BR, d), lambda i: (i, 0)),
            pl.BlockSpec((_BR, d), lambda i: (i, 0)),
            pl.BlockSpec((1, d), lambda i: (0, 0)),
            pl.BlockSpec((d, d), lambda i: (0, 0)),
        ],
        out_specs=pl.BlockSpec((_BR, d), lambda i: (i, 0)),
        out_shape=jax.ShapeDtypeStruct((n, d), jnp.float32),
    )(sp, g, dinvb, b, w)


def _tc2(sp, g, dinvb, b):
    n, d = g.shape

    def body(sp_ref, g_ref, dinv_ref, b_ref, out_ref):
        s = sp_ref[0] + sp_ref[1] + g_ref[...]
        out_ref[...] = jnp.maximum(dinv_ref[...] * s + b_ref[...], 0.0)

    return pl.pallas_call(
        body,
        grid=(n // _BR,),
        in_specs=[
            pl.BlockSpec((NC, _BR, d), lambda i: (0, i, 0)),
            pl.BlockSpec((_BR, d), lambda i: (i, 0)),
            pl.BlockSpec((_BR, d), lambda i: (i, 0)),
            pl.BlockSpec((1, d), lambda i: (0, 0)),
        ],
        out_specs=pl.BlockSpec((_BR, d), lambda i: (i, 0)),
        out_shape=jax.ShapeDtypeStruct((n, d), jnp.float32),
    )(sp, g, dinvb, b)


def kernel(x, edge_index, weights, W0, b0, W1, b1):
    n, d = x.shape
    e = edge_index.shape[1]
    npad = -(-n // _BR) * _BR
    assert e % (NW * K) == 0 and npad % (NS * 8) == 0

    ei = edge_index.astype(jnp.int32)
    row = ei[0]
    col = ei[1]
    ew = weights.astype(jnp.float32)
    xp = jnp.pad(x, ((0, npad - n), (0, 0)))

    degp = _sc_deg(col, ew, n=npad).reshape(NW, npad)
    g0, dinvb = _tc0(xp, W0, degp)
    s0 = _sc_msg(row, col, ew, g0)
    g1 = _tc1(s0, g0, dinvb, b0.reshape(1, d), W1)
    s1 = _sc_msg(row, col, ew, g1)
    out = _tc2(s1, g1, dinvb, b1.reshape(1, d))
    return out[:n]


# speedup vs baseline: 30.7476x; 3.4145x over previous
import dataclasses
import functools

import jax
import jax.numpy as jnp
from jax import lax
from jax.experimental import pallas as pl
from jax.experimental.pallas import tpu as pltpu
from jax.experimental.pallas import tpu_sc as plsc

NC = 2
NS = 16
NW = NC * NS
LANES = 16
K = 64
NBUF = 4


def _mesh():
    return plsc.VectorSubcoreMesh(core_axis_name="c", subcore_axis_name="s")


def _sc_compiler_params():
    cp = pltpu.CompilerParams()
    if "needs_layout_passes" in pltpu.CompilerParams.__dataclass_fields__:
        cp = dataclasses.replace(cp, needs_layout_passes=False)
    return cp


def _sc_deg(col, ew, *, n):
    e = col.shape[0]
    epw = e // NW
    ch = epw // K

    @functools.partial(
        pl.kernel,
        mesh=_mesh(),
        out_type=jax.ShapeDtypeStruct((NW * n,), jnp.float32),
        scratch_types=[
            pltpu.VMEM((epw,), jnp.int32),
            pltpu.VMEM((epw,), jnp.float32),
            pltpu.VMEM((n,), jnp.float32),
        ],
        compiler_params=_sc_compiler_params(),
    )
    def k(col_hbm, ew_hbm, out_hbm, colv, eww, dacc):
        cid = lax.axis_index("c")
        sid = lax.axis_index("s")
        wid = cid * NS + sid
        zero16 = jnp.zeros((LANES,), jnp.float32)

        @pl.loop(0, n // LANES)
        def _(i):
            dacc[pl.ds(i * LANES, LANES)] = zero16

        pltpu.sync_copy(col_hbm.at[pl.ds(wid * epw, epw)], colv)
        pltpu.sync_copy(ew_hbm.at[pl.ds(wid * epw, epw)], eww)

        @pl.loop(0, epw // LANES)
        def _(t):
            idx = colv[pl.ds(t * LANES, LANES)]
            v = eww[pl.ds(t * LANES, LANES)]
            plsc.addupdate_scatter(dacc, [idx], v)

        pltpu.sync_copy(dacc, out_hbm.at[pl.ds(wid * n, n)])

    return k(col, ew)


def _sc_msg(row, colr, ew, g):
    e = row.shape[0]
    _, ch, k_ = colr.shape
    assert k_ == K
    epw = e // NW
    n, d = g.shape
    rpt = n // NS
    assert n % NS == 0 and rpt % K == 0 and rpt % 8 == 0
    UR = 8
    assert K % UR == 0
    assert ch % NBUF == 0 and ch >= 2 * NBUF

    @functools.partial(
        pl.kernel,
        mesh=_mesh(),
        out_type=jax.ShapeDtypeStruct((NC, n, d), jnp.float32),
        scratch_types=[
            pltpu.VMEM((epw,), jnp.int32),
            pltpu.VMEM((NBUF, K), jnp.int32),
            pltpu.VMEM((NBUF, K), jnp.float32),
            pltpu.VMEM((NBUF, K, d), jnp.float32),
            pltpu.VMEM_SHARED((n, d), jnp.float32),
            pltpu.SemaphoreType.DMA,
        ] + [pltpu.SemaphoreType.DMA] * (4 * NBUF),
        compiler_params=_sc_compiler_params(),
    )
    def k(row_hbm, col_hbm, ew_hbm, g_hbm, out_hbm,
          rowa, colb, ewb, rowsb, acc, sem_p, *sems):
        cid = lax.axis_index("c")
        sid = lax.axis_index("s")
        wid = cid * NS + sid
        zero16 = jnp.zeros((LANES,), jnp.float32)
        sg = sems[0:NBUF]
        se = sems[NBUF:2 * NBUF]
        sc = sems[2 * NBUF:3 * NBUF]
        ss = sems[3 * NBUF:4 * NBUF]

        def issue_in(j, b):
            pltpu.async_copy(
                ew_hbm.at[pl.ds(wid * epw + j * K, K)], ewb.at[b], se[b])
            pltpu.async_copy(col_hbm.at[wid, j], colb.at[b], sc[b])
            pltpu.async_copy(
                g_hbm.at[rowa.at[pl.ds(j * K, K)]], rowsb.at[b], sg[b])

        def wait_in(j, b):
            pltpu.make_async_copy(
                ew_hbm.at[pl.ds(wid * epw + j * K, K)], ewb.at[b],
                se[b]).wait()
            pltpu.make_async_copy(
                g_hbm.at[rowa.at[pl.ds(j * K, K)]], rowsb.at[b],
                sg[b]).wait()

        def wait_col(j, b):
            pltpu.make_async_copy(
                col_hbm.at[wid, j], colb.at[b], sc[b]).wait()

        def issue_scatter(b):
            pltpu.async_copy(rowsb.at[b], acc.at[colb.at[b]], ss[b],
                             add=True)

        def wait_scatter(b):
            pltpu.make_async_copy(rowsb.at[b], acc.at[colb.at[b]],
                                  ss[b]).wait()

        def scale(b):
            @pl.loop(0, K, step=UR)
            def _(r0):
                for u in range(UR):
                    r = r0 + u
                    s = plsc.load_gather(
                        ewb.at[b], [jnp.full((LANES,), r, jnp.int32)])
                    for q in range(d // LANES):
                        rowsb[b, r, pl.ds(q * LANES, LANES)] = (
                            rowsb[b, r, pl.ds(q * LANES, LANES)] * s
                        )

        cp_r = pltpu.async_copy(row_hbm.at[pl.ds(wid * epw, epw)], rowa,
                                sem_p)

        @pl.loop(0, K)
        def _(i):
            for q in range(d // LANES):
                rowsb[0, i, pl.ds(q * LANES, LANES)] = zero16

        @pl.loop(0, rpt // K)
        def _(i):
            pltpu.sync_copy(rowsb.at[0],
                            acc.at[pl.ds(sid * rpt + i * K, K)])

        cp_r.wait()
        issue_in(jnp.int32(0), 0)
        issue_in(jnp.int32(1), 1)
        plsc.subcore_barrier()

        @pl.loop(0, ch, step=NBUF)
        def _(j0):
            for b in range(NBUF):
                jj = j0 + b

                wait_in(jj, b)

                @pl.when(jj >= 2)
                def _():
                    wait_scatter((b + 2) % NBUF)

                @pl.when(jj + 2 < ch)
                def _():
                    issue_in(jj + 2, (b + 2) % NBUF)

                scale(b)
                wait_col(jj, b)
                issue_scatter(b)

        wait_scatter((ch - 2) % NBUF)
        wait_scatter((ch - 1) % NBUF)
        plsc.subcore_barrier()
        pltpu.sync_copy(
            acc.at[pl.ds(sid * rpt, rpt)],
            out_hbm.at[cid, pl.ds(sid * rpt, rpt)],
        )

    return k(row, colr, ew, g)


_BR = 1024


def _tc0(x, w0, degp):
    n, d = x.shape

    def body(x_ref, w_ref, degp_ref, g_ref, dinv_ref):
        h = jnp.dot(x_ref[...], w_ref[...], preferred_element_type=jnp.float32)
        ones = jnp.ones((NW, 1), jnp.float32)
        dsum = 1.0 + lax.dot_general(
            degp_ref[...], ones, (((0,), (0,)), ((), ())),
            preferred_element_type=jnp.float32)
        dinv = jnp.where(dsum > 0, lax.rsqrt(dsum), 0.0)
        db = jnp.broadcast_to(dinv, (_BR, d))
        g_ref[...] = h * db
        dinv_ref[...] = db

    return pl.pallas_call(
        body,
        grid=(n // _BR,),
        in_specs=[
            pl.BlockSpec((_BR, d), lambda i: (i, 0)),
            pl.BlockSpec((d, d), lambda i: (0, 0)),
            pl.BlockSpec((NW, _BR), lambda i: (0, i)),
        ],
        out_specs=[
            pl.BlockSpec((_BR, d), lambda i: (i, 0)),
            pl.BlockSpec((_BR, d), lambda i: (i, 0)),
        ],
        out_shape=[
            jax.ShapeDtypeStruct((n, d), jnp.float32),
            jax.ShapeDtypeStruct((n, d), jnp.float32),
        ],
    )(x, w0, degp)


def _tc1(sp, g, dinvb, b, w):
    n, d = g.shape

    def body(sp_ref, g_ref, dinv_ref, b_ref, w_ref, out_ref):
        s = sp_ref[0] + sp_ref[1] + g_ref[...]
        x1 = jnp.maximum(dinv_ref[...] * s + b_ref[...], 0.0)
        h1 = jnp.dot(x1, w_ref[...], preferred_element_type=jnp.float32)
        out_ref[...] = dinv_ref[...] * h1

    return pl.pallas_call(
        body,
        grid=(n // _BR,),
        in_specs=[
            pl.BlockSpec((NC, _BR, d), lambda i: (0, i, 0)),
            pl.BlockSpec((_BR, d), lambda i: (i, 0)),
            pl.BlockSpec((_BR, d), lambda i: (i, 0)),
            pl.BlockSpec((1, d), lambda i: (0, 0)),
            pl.BlockSpec((d, d), lambda i: (0, 0)),
        ],
        out_specs=pl.BlockSpec((_BR, d), lambda i: (i, 0)),
        out_shape=jax.ShapeDtypeStruct((n, d), jnp.float32),
    )(sp, g, dinvb, b, w)


def _tc2(sp, g, dinvb, b):
    n, d = g.shape

    def body(sp_ref, g_ref, dinv_ref, b_ref, out_ref):
        s = sp_ref[0] + sp_ref[1] + g_ref[...]
        out_ref[...] = jnp.maximum(dinv_ref[...] * s + b_ref[...], 0.0)

    return pl.pallas_call(
        body,
        grid=(n // _BR,),
        in_specs=[
            pl.BlockSpec((NC, _BR, d), lambda i: (0, i, 0)),
            pl.BlockSpec((_BR, d), lambda i: (i, 0)),
            pl.BlockSpec((_BR, d), lambda i: (i, 0)),
            pl.BlockSpec((1, d), lambda i: (0, 0)),
        ],
        out_specs=pl.BlockSpec((_BR, d), lambda i: (i, 0)),
        out_shape=jax.ShapeDtypeStruct((n, d), jnp.float32),
    )(sp, g, dinvb, b)


def kernel(x, edge_index, weights, W0, b0, W1, b1):
    n, d = x.shape
    e = edge_index.shape[1]
    npad = -(-n // _BR) * _BR
    assert npad % (NS * 8) == 0

    ei = edge_index.astype(jnp.int32)
    ew = weights.astype(jnp.float32)
    xp = jnp.pad(x, ((0, npad - n), (0, 0)))
    epad = -(-e // (NW * NBUF * K)) * (NW * NBUF * K)
    pad = epad - e
    pad_src = jnp.arange(pad, dtype=jnp.int32) % n
    pad_dst = n + jnp.arange(pad, dtype=jnp.int32) % (npad - n)
    row = jnp.concatenate([ei[0], pad_src])
    col = jnp.concatenate([ei[1], pad_dst])
    ew = jnp.pad(ew, (0, pad))
    ch = epad // (NW * K)
    colr = col.reshape(NW, ch, K)

    degp = _sc_deg(col, ew, n=npad).reshape(NW, npad)
    g0, dinvb = _tc0(xp, W0, degp)
    s0 = _sc_msg(row, colr, ew, g0)
    g1 = _tc1(s0, g0, dinvb, b0.reshape(1, d), W1)
    s1 = _sc_msg(row, colr, ew, g1)
    out = _tc2(s1, g1, dinvb, b1.reshape(1, d))
    return out[:n]
